# baseline (speedup 1.0000x reference)
import functools

import jax
import jax.numpy as jnp
from jax import lax
from jax.experimental import pallas as pl
from jax.experimental.pallas import tpu as pltpu
from jax.experimental.pallas import tpu_sc as plsc

N_TOKEN = 1000000
CUTOFF = 100000
D_EMBED = 128
D_PROJ = 128
D_TAIL = 32

NC = 2
NS = 16
NW = NC * NS
LANES = 16

B_TOK = 1024 * 200
TOK_PER_W = B_TOK // NW
CHUNK = 128
N_CHUNK = TOK_PER_W // CHUNK
NBUF = 5
PF = NBUF - 1


def _sc_gather_one(inp_flat, table, l_idx, r_idx, d, tc_tiling):
    mesh = plsc.VectorSubcoreMesh(core_axis_name="c", subcore_axis_name="s")

    @functools.partial(
        pl.kernel,
        out_type=jax.ShapeDtypeStruct((B_TOK, d), jnp.float32),
        mesh=mesh,
        scratch_types=[
            pltpu.VMEM((TOK_PER_W,), jnp.int32),
            pltpu.VMEM((TOK_PER_W,), jnp.int32),
            [pltpu.VMEM((CHUNK, d), jnp.float32) for _ in range(NBUF)],
            [pltpu.SemaphoreType.DMA for _ in range(NBUF)],
            [pltpu.SemaphoreType.DMA for _ in range(NBUF)],
            pltpu.SemaphoreType.DMA,
        ],
        compiler_params=pltpu.CompilerParams(use_tc_tiling_on_sc=tc_tiling),
    )
    def k(inp_hbm, tab_hbm, x_hbm, inp_v, idx_v, x_v, gsem, wsem, isem):
        wid = lax.axis_index("s") * NC + lax.axis_index("c")
        w_base = wid * TOK_PER_W

        pltpu.make_async_copy(
            inp_hbm.at[pl.ds(w_base, TOK_PER_W)], inp_v, isem).start()
        pltpu.make_async_copy(
            inp_hbm.at[pl.ds(w_base, TOK_PER_W)], inp_v, isem).wait()

        def idx_body(g, _):
            for u in range(8):
                off = g * CHUNK + u * LANES
                t = inp_v[pl.ds(off, LANES)]
                i0 = jnp.minimum(jnp.maximum(t - l_idx, 0), r_idx - l_idx - 1)
                idx_v[pl.ds(off, LANES)] = i0
            return ()

        lax.fori_loop(0, N_CHUNK, idx_body, ())

        def g_copies(c, b):
            i0 = idx_v.at[pl.ds(c * CHUNK, CHUNK)]
            return (pltpu.make_async_copy(tab_hbm.at[i0], x_v[b], gsem[b]),)

        def w_copies(c, b):
            dst = pl.ds(w_base + c * CHUNK, CHUNK)
            return (pltpu.make_async_copy(x_v[b], x_hbm.at[dst], wsem[b]),)

        for b in range(PF):
            for cp in g_copies(b, b):
                cp.start()

        def body(g, _):
            for u in range(NBUF):
                c = g * NBUF + u
                nb = (u + PF) % NBUF
                for cp in g_copies(c, u):
                    cp.wait()
                for cp in w_copies(c, u):
                    cp.start()

                @pl.when(c + PF < N_CHUNK)
                def _():
                    @pl.when(c >= 1)
                    def _():
                        for cp in w_copies(c - 1, nb):
                            cp.wait()
                    for cp in g_copies(c + PF, nb):
                        cp.start()
            return ()

        lax.fori_loop(0, N_CHUNK // NBUF, body, ())

        for u in range(NBUF):
            c = N_CHUNK - NBUF + u
            for cp in w_copies(c, c % NBUF):
                cp.wait()

    return k(inp_flat, table)


def _tc_project(x0, x1, inp_flat, proj0, proj1):
    scale = float(D_PROJ) ** 0.5
    BT = 2048
    grid = (B_TOK // BT,)

    def body(inp_ref, x0_ref, x1_ref, p0_ref, p1_ref, out_ref):
        m = inp_ref[:] < CUTOFF
        y0 = lax.dot_general(x0_ref[:], p0_ref[:], (((1,), (1,)), ((), ())),
                             preferred_element_type=jnp.float32)
        y1 = lax.dot_general(x1_ref[:], p1_ref[:], (((1,), (1,)), ((), ())),
                             preferred_element_type=jnp.float32)
        out_ref[:] = jnp.where(m, y0, y1) * scale

    return pl.pallas_call(
        body,
        grid=grid,
        in_specs=[
            pl.BlockSpec((BT, 1), lambda i: (i, 0)),
            pl.BlockSpec((BT, D_EMBED), lambda i: (i, 0)),
            pl.BlockSpec((BT, D_TAIL), lambda i: (i, 0)),
            pl.BlockSpec((D_PROJ, D_EMBED), lambda i: (0, 0)),
            pl.BlockSpec((D_PROJ, D_TAIL), lambda i: (0, 0)),
        ],
        out_specs=pl.BlockSpec((BT, D_PROJ), lambda i: (i, 0)),
        out_shape=jax.ShapeDtypeStruct((B_TOK, D_PROJ), jnp.float32),
    )(inp_flat[:, None], x0, x1, proj0, proj1)


def kernel(inp, emb0, proj0, emb1, proj1):
    inp_flat = inp.reshape(-1).astype(jnp.int32)
    x0 = _sc_gather_one(inp_flat, emb0, 0, CUTOFF, D_EMBED, True)
    x1 = _sc_gather_one(inp_flat, emb1, CUTOFF, N_TOKEN, D_TAIL, False)
    out = _tc_project(x0, x1, inp_flat, proj0, proj1)
    return out.reshape(inp.shape + (D_PROJ,))


# speedup vs baseline: 10.9744x; 10.9744x over previous
import functools

import jax
import jax.numpy as jnp
from jax import lax
from jax.experimental import pallas as pl
from jax.experimental.pallas import tpu as pltpu
from jax.experimental.pallas import tpu_sc as plsc

N_TOKEN = 1000000
CUTOFF = 100000
D_EMBED = 128
D_PROJ = 128
D_TAIL = 32

NC = 2
NS = 16
NW = NC * NS
LANES = 16

B_TOK = 1024 * 200
TOK_PER_W = B_TOK // NW
CHUNK = 128
N_CHUNK = TOK_PER_W // CHUNK
NBUF = 5
PF = NBUF - 1


def _sc_gather_one(inp_flat, table, idx_fn, d, tc_tiling):
    mesh = plsc.VectorSubcoreMesh(core_axis_name="c", subcore_axis_name="s")

    @functools.partial(
        pl.kernel,
        out_type=jax.ShapeDtypeStruct((B_TOK, d), jnp.float32),
        mesh=mesh,
        scratch_types=[
            pltpu.VMEM((TOK_PER_W,), jnp.int32),
            pltpu.VMEM((TOK_PER_W,), jnp.int32),
            [pltpu.VMEM((CHUNK, d), jnp.float32) for _ in range(NBUF)],
            [pltpu.SemaphoreType.DMA for _ in range(NBUF)],
            [pltpu.SemaphoreType.DMA for _ in range(NBUF)],
            pltpu.SemaphoreType.DMA,
        ],
        compiler_params=pltpu.CompilerParams(use_tc_tiling_on_sc=tc_tiling),
    )
    def k(inp_hbm, tab_hbm, x_hbm, inp_v, idx_v, x_v, gsem, wsem, isem):
        wid = lax.axis_index("s") * NC + lax.axis_index("c")
        w_base = wid * TOK_PER_W

        pltpu.make_async_copy(
            inp_hbm.at[pl.ds(w_base, TOK_PER_W)], inp_v, isem).start()
        pltpu.make_async_copy(
            inp_hbm.at[pl.ds(w_base, TOK_PER_W)], inp_v, isem).wait()

        def idx_body(g, _):
            for u in range(8):
                off = g * CHUNK + u * LANES
                t = inp_v[pl.ds(off, LANES)]
                idx_v[pl.ds(off, LANES)] = idx_fn(t)
            return ()

        lax.fori_loop(0, N_CHUNK, idx_body, ())

        def g_copies(c, b):
            i0 = idx_v.at[pl.ds(c * CHUNK, CHUNK)]
            return (pltpu.make_async_copy(tab_hbm.at[i0], x_v[b], gsem[b]),)

        def w_copies(c, b):
            dst = pl.ds(w_base + c * CHUNK, CHUNK)
            return (pltpu.make_async_copy(x_v[b], x_hbm.at[dst], wsem[b]),)

        for b in range(PF):
            for cp in g_copies(b, b):
                cp.start()

        def body(g, _):
            for u in range(NBUF):
                c = g * NBUF + u
                nb = (u + PF) % NBUF
                for cp in g_copies(c, u):
                    cp.wait()
                for cp in w_copies(c, u):
                    cp.start()

                @pl.when(c + PF < N_CHUNK)
                def _():
                    @pl.when(c >= 1)
                    def _():
                        for cp in w_copies(c - 1, nb):
                            cp.wait()
                    for cp in g_copies(c + PF, nb):
                        cp.start()
            return ()

        lax.fori_loop(0, N_CHUNK // NBUF, body, ())

        for u in range(NBUF):
            c = N_CHUNK - NBUF + u
            for cp in w_copies(c, c % NBUF):
                cp.wait()

    return k(inp_flat, table)


def _tc_project(x0, x1, inp_flat, proj0, proj1):
    scale = float(D_PROJ) ** 0.5
    BT = 2048
    grid = (B_TOK // BT,)

    def body(inp_ref, x0_ref, x1_ref, p0_ref, p1_ref, out_ref):
        m = inp_ref[:] < CUTOFF
        y0 = lax.dot_general(x0_ref[:], p0_ref[:], (((1,), (1,)), ((), ())),
                             preferred_element_type=jnp.float32)
        y1 = lax.dot_general(x1_ref[:], p1_ref[:], (((1,), (1,)), ((), ())),
                             preferred_element_type=jnp.float32)
        out_ref[:] = jnp.where(m, y0, y1) * scale

    return pl.pallas_call(
        body,
        grid=grid,
        in_specs=[
            pl.BlockSpec((BT, 1), lambda i: (i, 0)),
            pl.BlockSpec((BT, D_EMBED), lambda i: (i, 0)),
            pl.BlockSpec((BT, D_TAIL), lambda i: (i, 0)),
            pl.BlockSpec((D_PROJ, D_EMBED), lambda i: (0, 0)),
            pl.BlockSpec((D_PROJ, D_TAIL), lambda i: (0, 0)),
        ],
        out_specs=pl.BlockSpec((BT, D_PROJ), lambda i: (i, 0)),
        out_shape=jax.ShapeDtypeStruct((B_TOK, D_PROJ), jnp.float32),
    )(inp_flat[:, None], x0, x1, proj0, proj1)


def _idx_head(t):
    i = jnp.where(t < CUTOFF, t, jnp.bitwise_and(t, 65535))
    return jnp.minimum(jnp.maximum(i, 0), CUTOFF - 1)


def _idx_tail(t):
    i = jnp.where(t >= CUTOFF, t - CUTOFF, t)
    return jnp.minimum(jnp.maximum(i, 0), N_TOKEN - CUTOFF - 1)


def kernel(inp, emb0, proj0, emb1, proj1):
    inp_flat = inp.reshape(-1).astype(jnp.int32)
    x0 = _sc_gather_one(inp_flat, emb0, _idx_head, D_EMBED, True)
    x1 = _sc_gather_one(inp_flat, emb1, _idx_tail, D_TAIL, False)
    out = _tc_project(x0, x1, inp_flat, proj0, proj1)
    return out.reshape(inp.shape + (D_PROJ,))
